# baseline (speedup 1.0000x reference)
import functools

import jax
import jax.numpy as jnp
from jax import lax
from jax.experimental import pallas as pl
from jax.experimental.pallas import tpu as pltpu
from jax.experimental.pallas import tpu_sc as plsc

_D = 32
_NC = 2
_NS = 16
_NW = _NC * _NS
_CHUNK = 128
_NBUF = 4
_NOBUF = 2


@functools.lru_cache(maxsize=None)
def _build_gather(n_rows: int):
    assert n_rows % (_NW * _CHUNK) == 0
    b_per_w = n_rows // _NW
    n_chunk = b_per_w // _CHUNK
    n_group = n_chunk // _NBUF
    mesh = plsc.VectorSubcoreMesh(
        core_axis_name="c", subcore_axis_name="s",
        num_cores=_NC, num_subcores=_NS)

    @functools.partial(
        pl.kernel,
        out_type=jax.ShapeDtypeStruct((n_rows, _D), jnp.float32),
        mesh=mesh,
        compiler_params=pltpu.CompilerParams(use_tc_tiling_on_sc=False),
        scratch_types=[
            pltpu.VMEM((b_per_w,), jnp.int32),
            pltpu.VMEM((_NBUF, _CHUNK, _D), jnp.float32),
        ] + [pltpu.SemaphoreType.DMA] * (2 * _NBUF),
    )
    def k_gather(table_hbm, idx_hbm, out_hbm, idx_v, gbuf,
                 g0, g1, g2, g3, o0, o1, o2, o3):
        gsems = (g0, g1, g2, g3)
        osems = (o0, o1, o2, o3)
        wid = lax.axis_index("s") * _NC + lax.axis_index("c")
        base = wid * b_per_w
        pltpu.sync_copy(idx_hbm.at[pl.ds(base, b_per_w)], idx_v)

        def fire_gather(c, u):
            pltpu.async_copy(
                table_hbm.at[idx_v.at[pl.ds(c * _CHUNK, _CHUNK)]],
                gbuf.at[u], gsems[u])

        for u in range(_NBUF - 1):
            fire_gather(u, u)

        def group_body(g, carry):
            for u in range(_NBUF):
                c = g * _NBUF + u
                v = (u - 1) % _NBUF
                pltpu.make_async_copy(
                    table_hbm.at[pl.ds(0, _CHUNK)], gbuf.at[u],
                    gsems[u]).wait()

                pltpu.async_copy(
                    gbuf.at[u],
                    out_hbm.at[pl.ds(base + c * _CHUNK, _CHUNK)],
                    osems[u])

                @pl.when(c + _NBUF - 1 < n_chunk)
                def _():
                    @pl.when(c >= 1)
                    def _():
                        pltpu.make_async_copy(
                            gbuf.at[v], out_hbm.at[pl.ds(0, _CHUNK)],
                            osems[v]).wait()
                    fire_gather(c + _NBUF - 1, v)
            return carry

        lax.fori_loop(0, n_group, group_body, 0)

        for u in range(_NBUF):
            pltpu.make_async_copy(
                gbuf.at[u], out_hbm.at[pl.ds(0, _CHUNK)], osems[u]).wait()

    return k_gather


@functools.lru_cache(maxsize=None)
def _build_pack(batch: int, fields: int):
    n_rows = batch * fields
    n_kb = batch // 128
    kb_per_w = n_kb // _NW
    blk_rows = 128 * fields
    blk_elems = blk_rows * _D
    n_kd = _D // 8
    n_tiles = fields * n_kd * n_kb
    stride = fields * _D
    mesh = plsc.VectorSubcoreMesh(
        core_axis_name="c", subcore_axis_name="s",
        num_cores=_NC, num_subcores=_NS)

    @functools.partial(
        pl.kernel,
        out_type=jax.ShapeDtypeStruct((n_tiles, 8, 128), jnp.float32),
        mesh=mesh,
        compiler_params=pltpu.CompilerParams(
            use_tc_tiling_on_sc=False, needs_layout_passes=False),
        scratch_types=[
            pltpu.VMEM((blk_elems,), jnp.float32),
            pltpu.VMEM((_NOBUF, n_kd, 8, 128), jnp.float32),
            pltpu.SemaphoreType.DMA,
            pltpu.SemaphoreType.DMA,
            pltpu.SemaphoreType.DMA,
        ],
    )
    def k_pack(g_hbm, out_hbm, blk_v, tbuf, insem, s0, s1):
        ssems = (s0, s1)
        wid = lax.axis_index("s") * _NC + lax.axis_index("c")
        iota_s = lax.iota(jnp.int32, 16) * stride

        for kb_l in range(kb_per_w):
            kb = wid * kb_per_w + kb_l
            pltpu.async_copy(
                g_hbm.at[pl.ds(kb * blk_elems, blk_elems)], blk_v,
                insem).wait()

            def f_body(f2, carry):
                for o in range(_NOBUF):
                    f = f2 * _NOBUF + o

                    @pl.when(f >= _NOBUF)
                    def _():
                        for _k in range(n_kd):
                            pltpu.make_async_copy(
                                tbuf.at[o, pl.ds(0, 1)],
                                out_hbm.at[pl.ds(0, 1)], ssems[o]).wait()

                    fvec = jnp.broadcast_to(f * _D, (16,)).astype(jnp.int32)
                    base = fvec + iota_s
                    for kd in range(n_kd):
                        for i in range(8):
                            for j0 in range(8):
                                offs = base + (j0 * 16 * stride
                                               + kd * 8 + i)
                                v = plsc.load_gather(blk_v, [offs])
                                tbuf[o, kd, i, pl.ds(j0 * 16, 16)] = v

                    row0 = f * (n_kd * n_kb) + kb
                    for kd in range(n_kd):
                        pltpu.async_copy(
                            tbuf.at[o, pl.ds(kd, 1)],
                            out_hbm.at[pl.ds(row0 + kd * n_kb, 1)],
                            ssems[o])
                return carry

            lax.fori_loop(0, fields // _NOBUF, f_body, 0)

            for o in range(_NOBUF):
                for _k in range(n_kd):
                    pltpu.make_async_copy(
                        tbuf.at[o, pl.ds(0, 1)],
                        out_hbm.at[pl.ds(0, 1)], ssems[o]).wait()

    return k_pack


def kernel(x, table):
    b, f = x.shape
    n_vocab, d = table.shape
    flat = x.reshape(b * f).astype(jnp.int32)
    g = _build_gather(b * f)(table, flat)
    out = _build_pack(b, f)(g.reshape(-1))
    out = out.reshape(f, d // 8, b // 128, 8, 128)
    out = out.transpose(2, 4, 0, 1, 3).reshape(b, f, d)
    return out


# speedup vs baseline: 1.1049x; 1.1049x over previous
import functools

import jax
import jax.numpy as jnp
from jax import lax
from jax.experimental import pallas as pl
from jax.experimental.pallas import tpu as pltpu
from jax.experimental.pallas import tpu_sc as plsc

_D = 32
_NC = 2
_NS = 16
_NW = _NC * _NS
_CHUNK = 128
_NBUF = 4
_NOBUF = 2


@functools.lru_cache(maxsize=None)
def _build_gather(n_rows: int):
    assert n_rows % (_NW * _CHUNK) == 0
    b_per_w = n_rows // _NW
    n_chunk = b_per_w // _CHUNK
    n_group = n_chunk // _NBUF
    mesh = plsc.VectorSubcoreMesh(
        core_axis_name="c", subcore_axis_name="s",
        num_cores=_NC, num_subcores=_NS)

    @functools.partial(
        pl.kernel,
        out_type=jax.ShapeDtypeStruct((n_rows, _D), jnp.float32),
        mesh=mesh,
        compiler_params=pltpu.CompilerParams(use_tc_tiling_on_sc=False),
        scratch_types=[
            pltpu.VMEM((b_per_w,), jnp.int32),
            pltpu.VMEM((_NBUF, _CHUNK, _D), jnp.float32),
        ] + [pltpu.SemaphoreType.DMA] * (2 * _NBUF),
    )
    def k_gather(table_hbm, idx_hbm, out_hbm, idx_v, gbuf,
                 g0, g1, g2, g3, o0, o1, o2, o3):
        gsems = (g0, g1, g2, g3)
        osems = (o0, o1, o2, o3)
        wid = lax.axis_index("s") * _NC + lax.axis_index("c")
        base = wid * b_per_w
        pltpu.sync_copy(idx_hbm.at[pl.ds(base, b_per_w)], idx_v)

        def fire_gather(c, u):
            pltpu.async_copy(
                table_hbm.at[idx_v.at[pl.ds(c * _CHUNK, _CHUNK)]],
                gbuf.at[u], gsems[u])

        for u in range(_NBUF - 1):
            fire_gather(u, u)

        def group_body(g, carry):
            for u in range(_NBUF):
                c = g * _NBUF + u
                v = (u - 1) % _NBUF
                pltpu.make_async_copy(
                    table_hbm.at[pl.ds(0, _CHUNK)], gbuf.at[u],
                    gsems[u]).wait()

                pltpu.async_copy(
                    gbuf.at[u],
                    out_hbm.at[pl.ds(base + c * _CHUNK, _CHUNK)],
                    osems[u])

                @pl.when(c + _NBUF - 1 < n_chunk)
                def _():
                    @pl.when(c >= 1)
                    def _():
                        pltpu.make_async_copy(
                            gbuf.at[v], out_hbm.at[pl.ds(0, _CHUNK)],
                            osems[v]).wait()
                    fire_gather(c + _NBUF - 1, v)
            return carry

        lax.fori_loop(0, n_group, group_body, 0)

        for u in range(_NBUF):
            pltpu.make_async_copy(
                gbuf.at[u], out_hbm.at[pl.ds(0, _CHUNK)], osems[u]).wait()

    return k_gather


@functools.lru_cache(maxsize=None)
def _build_pack(batch: int, fields: int):
    n_rows = batch * fields
    n_kb = batch // 128
    kb_per_w = n_kb // _NW
    blk_rows = 128 * fields
    blk_elems = blk_rows * _D
    n_kd = _D // 8
    n_tiles = fields * n_kd * n_kb
    stride = fields * _D
    mesh = plsc.VectorSubcoreMesh(
        core_axis_name="c", subcore_axis_name="s",
        num_cores=_NC, num_subcores=_NS)

    @functools.partial(
        pl.kernel,
        out_type=jax.ShapeDtypeStruct((n_tiles * 1024,), jnp.float32),
        mesh=mesh,
        compiler_params=pltpu.CompilerParams(
            use_tc_tiling_on_sc=False, needs_layout_passes=False),
        scratch_types=[
            pltpu.VMEM((blk_elems,), jnp.float32),
            pltpu.VMEM((_NOBUF * n_kd * 1024,), jnp.float32),
            pltpu.SemaphoreType.DMA,
            pltpu.SemaphoreType.DMA,
            pltpu.SemaphoreType.DMA,
        ],
    )
    def k_pack(g_hbm, out_hbm, blk_v, tbuf, insem, s0, s1):
        ssems = (s0, s1)
        wid = lax.axis_index("s") * _NC + lax.axis_index("c")
        iota128 = lax.iota(jnp.int32, 16) * 128

        for kb_l in range(kb_per_w):
            kb = wid * kb_per_w + kb_l
            pltpu.async_copy(
                g_hbm.at[pl.ds(kb * blk_elems, blk_elems)], blk_v,
                insem).wait()

            def f_body(f2, carry):
                for o in range(_NOBUF):
                    f = f2 * _NOBUF + o

                    @pl.when(f >= _NOBUF)
                    def _():
                        for _k in range(n_kd):
                            pltpu.make_async_copy(
                                tbuf.at[pl.ds(0, 1024)],
                                out_hbm.at[pl.ds(0, 1024)],
                                ssems[o]).wait()

                    fbase = f * _D
                    for j in range(128):
                        rb = fbase + j * stride
                        v0 = blk_v[pl.ds(rb, 16)]
                        v1 = blk_v[pl.ds(rb + 16, 16)]
                        plsc.store_scatter(
                            tbuf, [iota128 + (o * n_kd * 1024 + j)], v0)
                        plsc.store_scatter(
                            tbuf, [iota128 + (o * n_kd * 1024 + 2048 + j)],
                            v1)

                    row0 = f * (n_kd * n_kb) + kb
                    for kd in range(n_kd):
                        pltpu.async_copy(
                            tbuf.at[pl.ds((o * n_kd + kd) * 1024, 1024)],
                            out_hbm.at[pl.ds((row0 + kd * n_kb) * 1024,
                                             1024)],
                            ssems[o])
                return carry

            lax.fori_loop(0, fields // _NOBUF, f_body, 0)

            for o in range(_NOBUF):
                for _k in range(n_kd):
                    pltpu.make_async_copy(
                        tbuf.at[pl.ds(0, 1024)],
                        out_hbm.at[pl.ds(0, 1024)], ssems[o]).wait()

    return k_pack


def kernel(x, table):
    b, f = x.shape
    n_vocab, d = table.shape
    flat = x.reshape(b * f).astype(jnp.int32)
    g = _build_gather(b * f)(table, flat)
    out = _build_pack(b, f)(g.reshape(-1))
    out = out.reshape(f, d // 8, b // 128, 8, 128)
    out = out.transpose(2, 4, 0, 1, 3).reshape(b, f, d)
    return out


# speedup vs baseline: 1.2408x; 1.1230x over previous
import functools

import jax
import jax.numpy as jnp
from jax import lax
from jax.experimental import pallas as pl
from jax.experimental.pallas import tpu as pltpu
from jax.experimental.pallas import tpu_sc as plsc

_D = 32
_NC = 2
_NS = 16
_NW = _NC * _NS
_CHUNK = 128
_NBUF = 4
_NOBUF = 2


@functools.lru_cache(maxsize=None)
def _build_gather(n_rows: int):
    assert n_rows % (_NW * _CHUNK) == 0
    b_per_w = n_rows // _NW
    n_chunk = b_per_w // _CHUNK
    n_group = n_chunk // _NBUF
    mesh = plsc.VectorSubcoreMesh(
        core_axis_name="c", subcore_axis_name="s",
        num_cores=_NC, num_subcores=_NS)

    @functools.partial(
        pl.kernel,
        out_type=jax.ShapeDtypeStruct((n_rows, _D), jnp.float32),
        mesh=mesh,
        compiler_params=pltpu.CompilerParams(use_tc_tiling_on_sc=False),
        scratch_types=[
            pltpu.VMEM((b_per_w,), jnp.int32),
            pltpu.VMEM((_NBUF, _CHUNK, _D), jnp.float32),
        ] + [pltpu.SemaphoreType.DMA] * (2 * _NBUF),
    )
    def k_gather(table_hbm, idx_hbm, out_hbm, idx_v, gbuf,
                 g0, g1, g2, g3, o0, o1, o2, o3):
        gsems = (g0, g1, g2, g3)
        osems = (o0, o1, o2, o3)
        wid = lax.axis_index("s") * _NC + lax.axis_index("c")
        base = wid * b_per_w
        pltpu.sync_copy(idx_hbm.at[pl.ds(base, b_per_w)], idx_v)

        def fire_gather(c, u):
            pltpu.async_copy(
                table_hbm.at[idx_v.at[pl.ds(c * _CHUNK, _CHUNK)]],
                gbuf.at[u], gsems[u])

        for u in range(_NBUF - 1):
            fire_gather(u, u)

        def group_body(g, carry):
            for u in range(_NBUF):
                c = g * _NBUF + u
                v = (u - 1) % _NBUF
                pltpu.make_async_copy(
                    table_hbm.at[pl.ds(0, _CHUNK)], gbuf.at[u],
                    gsems[u]).wait()

                pltpu.async_copy(
                    gbuf.at[u],
                    out_hbm.at[pl.ds(base + c * _CHUNK, _CHUNK)],
                    osems[u])

                @pl.when(c + _NBUF - 1 < n_chunk)
                def _():
                    @pl.when(c >= 1)
                    def _():
                        pltpu.make_async_copy(
                            gbuf.at[v], out_hbm.at[pl.ds(0, _CHUNK)],
                            osems[v]).wait()
                    fire_gather(c + _NBUF - 1, v)
            return carry

        lax.fori_loop(0, n_group, group_body, 0)

        for u in range(_NBUF):
            pltpu.make_async_copy(
                gbuf.at[u], out_hbm.at[pl.ds(0, _CHUNK)], osems[u]).wait()

    return k_gather


@functools.lru_cache(maxsize=None)
def _build_pack(batch: int, fields: int):
    n_rows = batch * fields
    n_kb = batch // 128
    kb_per_w = n_kb // _NW
    blk_rows = 128 * fields
    blk_elems = blk_rows * _D
    n_kd = _D // 8
    n_tiles = fields * n_kd * n_kb
    stride = fields * _D
    mesh = plsc.VectorSubcoreMesh(
        core_axis_name="c", subcore_axis_name="s",
        num_cores=_NC, num_subcores=_NS)

    @functools.partial(
        pl.kernel,
        out_type=jax.ShapeDtypeStruct((n_tiles * 1024,), jnp.float32),
        mesh=mesh,
        compiler_params=pltpu.CompilerParams(
            use_tc_tiling_on_sc=False, needs_layout_passes=False),
        scratch_types=[
            pltpu.VMEM((blk_elems,), jnp.float32),
            pltpu.VMEM((_D * 129,), jnp.float32),
            pltpu.VMEM((_NOBUF * n_kd * 1024,), jnp.float32),
            pltpu.SemaphoreType.DMA,
            pltpu.SemaphoreType.DMA,
            pltpu.SemaphoreType.DMA,
        ],
    )
    def k_pack(g_hbm, out_hbm, blk_v, pad_v, tbuf, insem, s0, s1):
        ssems = (s0, s1)
        wid = lax.axis_index("s") * _NC + lax.axis_index("c")
        iota129 = lax.iota(jnp.int32, 16) * 129

        for kb_l in range(kb_per_w):
            kb = wid * kb_per_w + kb_l
            pltpu.async_copy(
                g_hbm.at[pl.ds(kb * blk_elems, blk_elems)], blk_v,
                insem).wait()

            def f_body(f2, carry):
                for o in range(_NOBUF):
                    f = f2 * _NOBUF + o

                    @pl.when(f >= _NOBUF)
                    def _():
                        for _k in range(n_kd):
                            pltpu.make_async_copy(
                                tbuf.at[pl.ds(0, 1024)],
                                out_hbm.at[pl.ds(0, 1024)],
                                ssems[o]).wait()

                    fbase = f * _D
                    for j in range(128):
                        rb = fbase + j * stride
                        v0 = blk_v[pl.ds(rb, 16)]
                        v1 = blk_v[pl.ds(rb + 16, 16)]
                        plsc.store_scatter(pad_v, [iota129 + j], v0)
                        plsc.store_scatter(
                            pad_v, [iota129 + (16 * 129 + j)], v1)

                    for dd in range(_D):
                        for h in range(8):
                            tbuf[pl.ds(o * n_kd * 1024 + dd * 128 + h * 16,
                                       16)] = (
                                pad_v[pl.ds(dd * 129 + h * 16, 16)])

                    row0 = f * (n_kd * n_kb) + kb
                    for kd in range(n_kd):
                        pltpu.async_copy(
                            tbuf.at[pl.ds((o * n_kd + kd) * 1024, 1024)],
                            out_hbm.at[pl.ds((row0 + kd * n_kb) * 1024,
                                             1024)],
                            ssems[o])
                return carry

            lax.fori_loop(0, fields // _NOBUF, f_body, 0)

            for o in range(_NOBUF):
                for _k in range(n_kd):
                    pltpu.make_async_copy(
                        tbuf.at[pl.ds(0, 1024)],
                        out_hbm.at[pl.ds(0, 1024)], ssems[o]).wait()

    return k_pack


def kernel(x, table):
    b, f = x.shape
    n_vocab, d = table.shape
    flat = x.reshape(b * f).astype(jnp.int32)
    g = _build_gather(b * f)(table, flat)
    out = _build_pack(b, f)(g.reshape(-1))
    out = out.reshape(f, d // 8, b // 128, 8, 128)
    out = out.transpose(2, 4, 0, 1, 3).reshape(b, f, d)
    return out


# speedup vs baseline: 1.5086x; 1.2158x over previous
import functools

import jax
import jax.numpy as jnp
from jax import lax
from jax.experimental import pallas as pl
from jax.experimental.pallas import tpu as pltpu
from jax.experimental.pallas import tpu_sc as plsc

_D = 32
_NC = 2
_NS = 16
_NW = _NC * _NS
_CHUNK = 128
_NBUF = 4
_NOBUF = 2


@functools.lru_cache(maxsize=None)
def _build_gather(n_rows: int):
    assert n_rows % (_NW * _CHUNK) == 0
    b_per_w = n_rows // _NW
    n_chunk = b_per_w // _CHUNK
    n_group = n_chunk // _NBUF
    mesh = plsc.VectorSubcoreMesh(
        core_axis_name="c", subcore_axis_name="s",
        num_cores=_NC, num_subcores=_NS)

    @functools.partial(
        pl.kernel,
        out_type=jax.ShapeDtypeStruct((n_rows, _D), jnp.float32),
        mesh=mesh,
        compiler_params=pltpu.CompilerParams(use_tc_tiling_on_sc=False),
        scratch_types=[
            pltpu.VMEM((b_per_w,), jnp.int32),
            pltpu.VMEM((_NBUF, _CHUNK, _D), jnp.float32),
        ] + [pltpu.SemaphoreType.DMA] * (2 * _NBUF),
    )
    def k_gather(table_hbm, idx_hbm, out_hbm, idx_v, gbuf,
                 g0, g1, g2, g3, o0, o1, o2, o3):
        gsems = (g0, g1, g2, g3)
        osems = (o0, o1, o2, o3)
        wid = lax.axis_index("s") * _NC + lax.axis_index("c")
        base = wid * b_per_w
        pltpu.sync_copy(idx_hbm.at[pl.ds(base, b_per_w)], idx_v)

        def fire_gather(c, u):
            pltpu.async_copy(
                table_hbm.at[idx_v.at[pl.ds(c * _CHUNK, _CHUNK)]],
                gbuf.at[u], gsems[u])

        for u in range(_NBUF - 1):
            fire_gather(u, u)

        def group_body(g, carry):
            for u in range(_NBUF):
                c = g * _NBUF + u
                v = (u - 1) % _NBUF
                pltpu.make_async_copy(
                    table_hbm.at[pl.ds(0, _CHUNK)], gbuf.at[u],
                    gsems[u]).wait()

                pltpu.async_copy(
                    gbuf.at[u],
                    out_hbm.at[pl.ds(base + c * _CHUNK, _CHUNK)],
                    osems[u])

                @pl.when(c + _NBUF - 1 < n_chunk)
                def _():
                    @pl.when(c >= 1)
                    def _():
                        pltpu.make_async_copy(
                            gbuf.at[v], out_hbm.at[pl.ds(0, _CHUNK)],
                            osems[v]).wait()
                    fire_gather(c + _NBUF - 1, v)
            return carry

        lax.fori_loop(0, n_group, group_body, 0)

        for u in range(_NBUF):
            pltpu.make_async_copy(
                gbuf.at[u], out_hbm.at[pl.ds(0, _CHUNK)], osems[u]).wait()

    return k_gather


@functools.lru_cache(maxsize=None)
def _build_pack(batch: int, fields: int):
    n_rows = batch * fields
    n_kb = batch // 128
    kb_per_w = n_kb // _NW
    blk_rows = 128 * fields
    blk_elems = blk_rows * _D
    n_kd = _D // 8
    n_tiles = fields * n_kd * n_kb
    stride = fields * _D
    mesh = plsc.VectorSubcoreMesh(
        core_axis_name="c", subcore_axis_name="s",
        num_cores=_NC, num_subcores=_NS)

    @functools.partial(
        pl.kernel,
        out_type=jax.ShapeDtypeStruct((n_tiles * 1024,), jnp.float32),
        mesh=mesh,
        compiler_params=pltpu.CompilerParams(
            use_tc_tiling_on_sc=False, needs_layout_passes=False),
        scratch_types=[
            pltpu.VMEM((blk_elems,), jnp.float32),
            pltpu.VMEM((_D * 129,), jnp.float32),
            pltpu.VMEM((_NOBUF * n_kd * 1024,), jnp.float32),
            pltpu.SemaphoreType.DMA,
            pltpu.SemaphoreType.DMA,
            pltpu.SemaphoreType.DMA,
        ],
    )
    def k_pack(g_hbm, out_hbm, blk_v, pad_v, tbuf, insem, s0, s1):
        ssems = (s0, s1)
        wid = lax.axis_index("s") * _NC + lax.axis_index("c")
        iota129 = lax.iota(jnp.int32, 16) * 129

        for kb_l in range(kb_per_w):
            kb = wid * kb_per_w + kb_l
            pltpu.async_copy(
                g_hbm.at[pl.ds(kb * blk_elems, blk_elems)], blk_v,
                insem).wait()

            def f_body(f2, carry):
                for o in range(_NOBUF):
                    f = f2 * _NOBUF + o

                    @pl.when(f >= _NOBUF)
                    def _():
                        for _k in range(n_kd):
                            pltpu.make_async_copy(
                                tbuf.at[pl.ds(0, 1024)],
                                out_hbm.at[pl.ds(0, 1024)],
                                ssems[o]).wait()

                    fbase = f * _D

                    @plsc.parallel_loop(0, 128, unroll=8)
                    def _(j):
                        rb = fbase + j * stride
                        v0 = blk_v[pl.ds(rb, 16)]
                        v1 = blk_v[pl.ds(rb + 16, 16)]
                        plsc.store_scatter(pad_v, [iota129 + j], v0)
                        plsc.store_scatter(
                            pad_v, [iota129 + (16 * 129 + j)], v1)

                    @plsc.parallel_loop(0, _D, unroll=4)
                    def _(dd):
                        for h in range(8):
                            tbuf[pl.ds(o * n_kd * 1024 + dd * 128 + h * 16,
                                       16)] = (
                                pad_v[pl.ds(dd * 129 + h * 16, 16)])

                    row0 = f * (n_kd * n_kb) + kb
                    for kd in range(n_kd):
                        pltpu.async_copy(
                            tbuf.at[pl.ds((o * n_kd + kd) * 1024, 1024)],
                            out_hbm.at[pl.ds((row0 + kd * n_kb) * 1024,
                                             1024)],
                            ssems[o])
                return carry

            lax.fori_loop(0, fields // _NOBUF, f_body, 0)

            for o in range(_NOBUF):
                for _k in range(n_kd):
                    pltpu.make_async_copy(
                        tbuf.at[pl.ds(0, 1024)],
                        out_hbm.at[pl.ds(0, 1024)], ssems[o]).wait()

    return k_pack


def kernel(x, table):
    b, f = x.shape
    n_vocab, d = table.shape
    flat = x.reshape(b * f).astype(jnp.int32)
    g = _build_gather(b * f)(table, flat)
    out = _build_pack(b, f)(g.reshape(-1))
    out = out.reshape(f, d // 8, b // 128, 8, 128)
    out = out.transpose(2, 4, 0, 1, 3).reshape(b, f, d)
    return out
